# Initial kernel scaffold; baseline (speedup 1.0000x reference)
import functools

import jax
import jax.numpy as jnp
from jax import lax
from jax.experimental import pallas as pl
from jax.experimental.pallas import tpu as pltpu
from jax.experimental.pallas import tpu_sc as plsc

N = 10000
D = 128
N_PAD = 10240
CHUNK = 128
ROWS_PT = N_PAD // 16
NC, NS = 2, 16

_mesh = plsc.VectorSubcoreMesh(core_axis_name="c", subcore_axis_name="s")


def _deg_body(dst_hbm, ones_hbm, zer_hbm, out_hbm, dst_v, ones_v, buf_v, acc, sem):
    cid = lax.axis_index("c")
    sid = lax.axis_index("s")
    wid = cid * NS + sid
    e_pt = dst_hbm.shape[0] // (NC * NS)
    nch = e_pt // CHUNK
    r0 = sid * ROWS_PT
    pltpu.sync_copy(zer_hbm, buf_v)
    pltpu.sync_copy(buf_v, acc.at[pl.ds(r0, ROWS_PT)])
    pltpu.sync_copy(ones_hbm, ones_v)
    plsc.subcore_barrier()

    def body(c, carry):
        off = wid * e_pt + c * CHUNK
        pltpu.sync_copy(dst_hbm.at[pl.ds(off, CHUNK)], dst_v)
        pltpu.sync_copy(ones_v, acc.at[dst_v], add=True)
        return carry

    lax.fori_loop(0, nch, body, 0)
    plsc.subcore_barrier()
    pltpu.sync_copy(acc.at[pl.ds(r0, ROWS_PT)], buf_v)
    pltpu.sync_copy(buf_v, out_hbm.at[cid, pl.ds(r0, ROWS_PT)])


@functools.partial(
    pl.kernel,
    out_type=jax.ShapeDtypeStruct((NC, N_PAD, 16), jnp.float32),
    mesh=_mesh,
    scratch_types=[
        pltpu.VMEM((CHUNK,), jnp.int32),
        pltpu.VMEM((CHUNK, 16), jnp.float32),
        pltpu.VMEM((ROWS_PT, 16), jnp.float32),
        pltpu.VMEM_SHARED((N_PAD, 16), jnp.float32),
        pltpu.SemaphoreType.DMA,
    ],
)
def _deg_kernel(dst_hbm, ones_hbm, zer_hbm, out_hbm, dst_v, ones_v, buf_v, acc, sem):
    _deg_body(dst_hbm, ones_hbm, zer_hbm, out_hbm, dst_v, ones_v, buf_v, acc, sem)


def _seg_body(y_hbm, src_hbm, dst_hbm, zer_hbm, out_hbm, src_v, dst_v, rows_v, acc, sem):
    cid = lax.axis_index("c")
    sid = lax.axis_index("s")
    wid = cid * NS + sid
    e_pt = src_hbm.shape[0] // (NC * NS)
    nch = e_pt // CHUNK
    r0 = sid * ROWS_PT
    pltpu.sync_copy(zer_hbm, rows_v)
    for k in range(ROWS_PT // CHUNK):
        pltpu.sync_copy(rows_v, acc.at[pl.ds(r0 + k * CHUNK, CHUNK)])
    plsc.subcore_barrier()

    def body(c, carry):
        off = wid * e_pt + c * CHUNK
        pltpu.sync_copy(src_hbm.at[pl.ds(off, CHUNK)], src_v)
        pltpu.sync_copy(dst_hbm.at[pl.ds(off, CHUNK)], dst_v)
        pltpu.async_copy(y_hbm.at[src_v], rows_v, sem).wait()
        pltpu.sync_copy(rows_v, acc.at[dst_v], add=True)
        return carry

    lax.fori_loop(0, nch, body, 0)
    plsc.subcore_barrier()
    for k in range(ROWS_PT // CHUNK):
        pltpu.sync_copy(acc.at[pl.ds(r0 + k * CHUNK, CHUNK)], rows_v)
        pltpu.sync_copy(rows_v, out_hbm.at[cid, pl.ds(r0 + k * CHUNK, CHUNK)])


@functools.partial(
    pl.kernel,
    out_type=jax.ShapeDtypeStruct((NC, N_PAD, D), jnp.float32),
    mesh=_mesh,
    scratch_types=[
        pltpu.VMEM((CHUNK,), jnp.int32),
        pltpu.VMEM((CHUNK,), jnp.int32),
        pltpu.VMEM((CHUNK, D), jnp.float32),
        pltpu.VMEM_SHARED((N_PAD, D), jnp.float32),
        pltpu.SemaphoreType.DMA,
    ],
)
def _seg_kernel(y_hbm, src_hbm, dst_hbm, zer_hbm, out_hbm, src_v, dst_v, rows_v, acc, sem):
    _seg_body(y_hbm, src_hbm, dst_hbm, zer_hbm, out_hbm, src_v, dst_v, rows_v, acc, sem)


_BLK = 1024
_GRID = N_PAD // _BLK


def _mm1_body(x_ref, w_ref, degp_ref, y_ref, dinv_ref):
    degp = degp_ref[...]
    deg = degp[0, :, 0] + degp[1, :, 0] + 1.0
    dinv = lax.rsqrt(deg)
    y = jnp.dot(x_ref[...], w_ref[...], preferred_element_type=jnp.float32)
    y_ref[...] = y * dinv[:, None]
    dinv_ref[...] = dinv


def _mm1(x, w1, degp):
    return pl.pallas_call(
        _mm1_body,
        grid=(_GRID,),
        in_specs=[
            pl.BlockSpec((_BLK, D), lambda i: (i, 0)),
            pl.BlockSpec((D, D), lambda i: (0, 0)),
            pl.BlockSpec((NC, _BLK, 16), lambda i: (0, i, 0)),
        ],
        out_specs=[
            pl.BlockSpec((_BLK, D), lambda i: (i, 0)),
            pl.BlockSpec((_BLK,), lambda i: (i,)),
        ],
        out_shape=[
            jax.ShapeDtypeStruct((N_PAD, D), jnp.float32),
            jax.ShapeDtypeStruct((N_PAD,), jnp.float32),
        ],
    )(x, w1, degp)


def _mm2_body(sp_ref, y1_ref, dinv_ref, b1_ref, w2_ref, y2_ref):
    sp = sp_ref[...]
    dinv = dinv_ref[...]
    s = sp[0] + sp[1] + y1_ref[...]
    h = jnp.maximum(s * dinv[:, None] + b1_ref[...][None, :], 0.0)
    y2 = jnp.dot(h, w2_ref[...], preferred_element_type=jnp.float32)
    y2_ref[...] = y2 * dinv[:, None]


def _mm2(sp, y1, dinv, b1, w2):
    return pl.pallas_call(
        _mm2_body,
        grid=(_GRID,),
        in_specs=[
            pl.BlockSpec((NC, _BLK, D), lambda i: (0, i, 0)),
            pl.BlockSpec((_BLK, D), lambda i: (i, 0)),
            pl.BlockSpec((_BLK,), lambda i: (i,)),
            pl.BlockSpec((D,), lambda i: (0,)),
            pl.BlockSpec((D, D), lambda i: (0, 0)),
        ],
        out_specs=pl.BlockSpec((_BLK, D), lambda i: (i, 0)),
        out_shape=jax.ShapeDtypeStruct((N_PAD, D), jnp.float32),
    )(sp, y1, dinv, b1, w2)


def _out_body(sp_ref, y2_ref, dinv_ref, b2_ref, o_ref):
    sp = sp_ref[...]
    dinv = dinv_ref[...]
    s = (sp[0] + sp[1] + y2_ref[...]) * dinv[:, None] + b2_ref[...][None, :]
    m = jnp.max(s, axis=1, keepdims=True)
    lse = m + jnp.log(jnp.sum(jnp.exp(s - m), axis=1, keepdims=True))
    o_ref[...] = s - lse


def _outk(sp, y2, dinv, b2):
    return pl.pallas_call(
        _out_body,
        grid=(_GRID,),
        in_specs=[
            pl.BlockSpec((NC, _BLK, D), lambda i: (0, i, 0)),
            pl.BlockSpec((_BLK, D), lambda i: (i, 0)),
            pl.BlockSpec((_BLK,), lambda i: (i,)),
            pl.BlockSpec((D,), lambda i: (0,)),
        ],
        out_specs=pl.BlockSpec((_BLK, D), lambda i: (i, 0)),
        out_shape=jax.ShapeDtypeStruct((N_PAD, D), jnp.float32),
    )(sp, y2, dinv, b2)


def kernel(x, edge_index, W1, b1, W2, b2):
    e = edge_index.shape[1]
    e_pad = ((e + (NC * NS * CHUNK) - 1) // (NC * NS * CHUNK)) * (NC * NS * CHUNK)
    src = edge_index[0].astype(jnp.int32)
    dst = edge_index[1].astype(jnp.int32)
    src = jnp.concatenate([src, jnp.zeros((e_pad - e,), jnp.int32)])
    dst = jnp.concatenate([dst, jnp.full((e_pad - e,), N, jnp.int32)])
    x_pad = jnp.concatenate(
        [x.astype(jnp.float32), jnp.zeros((N_PAD - N, D), jnp.float32)]
    )
    ones16 = jnp.ones((CHUNK, 16), jnp.float32)
    zer16 = jnp.zeros((ROWS_PT, 16), jnp.float32)
    zer128 = jnp.zeros((CHUNK, D), jnp.float32)

    degp = _deg_kernel(dst, ones16, zer16)
    y1, dinv = _mm1(x_pad, W1.astype(jnp.float32), degp)
    s1p = _seg_kernel(y1, src, dst, zer128)
    y2 = _mm2(s1p, y1, dinv, b1.astype(jnp.float32), W2.astype(jnp.float32))
    s2p = _seg_kernel(y2, src, dst, zer128)
    out = _outk(s2p, y2, dinv, b2.astype(jnp.float32))
    return out[:N]


# speedup vs baseline: 5.9462x; 5.9462x over previous
import functools

import jax
import jax.numpy as jnp
from jax import lax
from jax.experimental import pallas as pl
from jax.experimental.pallas import tpu as pltpu
from jax.experimental.pallas import tpu_sc as plsc

N = 10000
D = 128
N_PAD = 10240
SUB = 16
CHUNK = 128
ROWS_PT = N_PAD // 16
NC, NS = 2, 16

_mesh = plsc.VectorSubcoreMesh(
    core_axis_name="c", subcore_axis_name="s", num_cores=NC, num_subcores=NS
)


def _deg_body(dst_hbm, iota_hbm, ones_hbm, zer_hbm, out_hbm,
              dst_v, idx_v, ones_v, zbuf_v, buf_v, acc, sem):
    cid = lax.axis_index("c")
    sid = lax.axis_index("s")
    wid = cid * NS + sid
    e_rows = dst_hbm.shape[0] // (NC * NS)
    nch = e_rows // (CHUNK // SUB)
    nz = ROWS_PT // CHUNK
    pltpu.async_copy(zer_hbm, zbuf_v, sem).wait()
    pltpu.async_copy(ones_hbm, ones_v, sem).wait()
    for k in range(nz):
        pltpu.async_copy(
            iota_hbm.at[pl.ds(sid * (ROWS_PT // SUB) + k * 8, 8)], idx_v, sem
        ).wait()
        for j in range(CHUNK // SUB):
            pltpu.async_copy(zbuf_v, acc.at[idx_v.at[j]], sem).wait()
    plsc.subcore_barrier()

    def body(c, carry):
        off = wid * e_rows + c * (CHUNK // SUB)
        pltpu.async_copy(dst_hbm.at[pl.ds(off, CHUNK // SUB)], dst_v, sem).wait()
        for j in range(CHUNK // SUB):
            pltpu.async_copy(ones_v, acc.at[dst_v.at[j]], sem, add=True).wait()
        return carry

    lax.fori_loop(0, nch, body, 0)
    plsc.subcore_barrier()
    for k in range(nz):
        pltpu.async_copy(
            iota_hbm.at[pl.ds(sid * (ROWS_PT // SUB) + k * 8, 8)], idx_v, sem
        ).wait()
        for j in range(CHUNK // SUB):
            pltpu.async_copy(acc.at[idx_v.at[j]],
                             buf_v.at[pl.ds(k * CHUNK + j * SUB, SUB)], sem).wait()
    pltpu.async_copy(
        buf_v, out_hbm.at[pl.ds(cid * N_PAD + sid * ROWS_PT, ROWS_PT)], sem
    ).wait()


@functools.partial(
    pl.kernel,
    out_type=jax.ShapeDtypeStruct((NC * N_PAD, 16), jnp.float32),
    mesh=_mesh,
    scratch_types=[
        pltpu.VMEM((CHUNK // SUB, SUB), jnp.int32),
        pltpu.VMEM((CHUNK // SUB, SUB), jnp.int32),
        pltpu.VMEM((SUB, 16), jnp.float32),
        pltpu.VMEM((SUB, 16), jnp.float32),
        pltpu.VMEM((ROWS_PT, 16), jnp.float32),
        pltpu.VMEM_SHARED((N_PAD, 16), jnp.float32),
        pltpu.SemaphoreType.DMA,
    ],
    compiler_params=pltpu.CompilerParams(use_tc_tiling_on_sc=False),
)
def _deg_kernel(dst_hbm, iota_hbm, ones_hbm, zer_hbm, out_hbm,
                dst_v, idx_v, ones_v, zbuf_v, buf_v, acc, sem):
    _deg_body(dst_hbm, iota_hbm, ones_hbm, zer_hbm, out_hbm,
              dst_v, idx_v, ones_v, zbuf_v, buf_v, acc, sem)


def _seg_body(y_hbm, src_hbm, dst_hbm, iota_hbm, zer_hbm, out_hbm,
              src_v, dst_v, idx_v, rows_v, zbuf_v, buf_v, acc, sem):
    cid = lax.axis_index("c")
    sid = lax.axis_index("s")
    wid = cid * NS + sid
    e_rows = src_hbm.shape[0] // (NC * NS)
    nch = e_rows // (CHUNK // SUB)
    nz = ROWS_PT // CHUNK
    pltpu.sync_copy(zer_hbm, zbuf_v)
    for k in range(nz):
        pltpu.sync_copy(iota_hbm.at[pl.ds(sid * (ROWS_PT // SUB) + k * 8, 8)], idx_v)
        for j in range(CHUNK // SUB):
            pltpu.sync_copy(zbuf_v, acc.at[idx_v.at[j]])
    plsc.subcore_barrier()

    def body(c, carry):
        off = wid * e_rows + c * (CHUNK // SUB)
        pltpu.sync_copy(src_hbm.at[pl.ds(off, CHUNK // SUB)], src_v)
        pltpu.sync_copy(dst_hbm.at[pl.ds(off, CHUNK // SUB)], dst_v)
        for j in range(CHUNK // SUB):
            pltpu.sync_copy(y_hbm.at[src_v.at[j]], rows_v)
            pltpu.sync_copy(rows_v, acc.at[dst_v.at[j]], add=True)
        return carry

    lax.fori_loop(0, nch, body, 0)
    plsc.subcore_barrier()
    for k in range(nz):
        pltpu.sync_copy(iota_hbm.at[pl.ds(sid * (ROWS_PT // SUB) + k * 8, 8)], idx_v)
        for j in range(CHUNK // SUB):
            pltpu.sync_copy(acc.at[idx_v.at[j]],
                            buf_v.at[pl.ds(j * SUB, SUB)])
        pltpu.sync_copy(
            buf_v,
            out_hbm.at[pl.ds(cid * N_PAD + sid * ROWS_PT + k * CHUNK, CHUNK)],
        )


@functools.partial(
    pl.kernel,
    out_type=jax.ShapeDtypeStruct((NC * N_PAD, D), jnp.float32),
    mesh=_mesh,
    scratch_types=[
        pltpu.VMEM((CHUNK // SUB, SUB), jnp.int32),
        pltpu.VMEM((CHUNK // SUB, SUB), jnp.int32),
        pltpu.VMEM((CHUNK // SUB, SUB), jnp.int32),
        pltpu.VMEM((SUB, D), jnp.float32),
        pltpu.VMEM((SUB, D), jnp.float32),
        pltpu.VMEM((CHUNK, D), jnp.float32),
        pltpu.VMEM_SHARED((N_PAD, D), jnp.float32),
        pltpu.SemaphoreType.DMA,
    ],
)
def _seg_kernel(y_hbm, src_hbm, dst_hbm, iota_hbm, zer_hbm, out_hbm,
                src_v, dst_v, idx_v, rows_v, zbuf_v, buf_v, acc, sem):
    _seg_body(y_hbm, src_hbm, dst_hbm, iota_hbm, zer_hbm, out_hbm,
              src_v, dst_v, idx_v, rows_v, zbuf_v, buf_v, acc, sem)


_BLK = 1024
_GRID = N_PAD // _BLK


def _mm1_body(x_ref, w_ref, degp_ref, y_ref, dinv_ref):
    degp = degp_ref[...]
    deg = degp[0, :, 0] + degp[1, :, 0] + 1.0
    dinv = lax.rsqrt(deg)
    y = jnp.dot(x_ref[...], w_ref[...], preferred_element_type=jnp.float32)
    y_ref[...] = y * dinv[:, None]
    dinv_ref[...] = dinv


def _mm1(x, w1, degp):
    return pl.pallas_call(
        _mm1_body,
        grid=(_GRID,),
        in_specs=[
            pl.BlockSpec((_BLK, D), lambda i: (i, 0)),
            pl.BlockSpec((D, D), lambda i: (0, 0)),
            pl.BlockSpec((NC, _BLK, 16), lambda i: (0, i, 0)),
        ],
        out_specs=[
            pl.BlockSpec((_BLK, D), lambda i: (i, 0)),
            pl.BlockSpec((_BLK,), lambda i: (i,)),
        ],
        out_shape=[
            jax.ShapeDtypeStruct((N_PAD, D), jnp.float32),
            jax.ShapeDtypeStruct((N_PAD,), jnp.float32),
        ],
    )(x, w1, degp)


def _mm2_body(sp_ref, y1_ref, dinv_ref, b1_ref, w2_ref, y2_ref):
    sp = sp_ref[...]
    dinv = dinv_ref[...]
    s = sp[0] + sp[1] + y1_ref[...]
    h = jnp.maximum(s * dinv[:, None] + b1_ref[...][None, :], 0.0)
    y2 = jnp.dot(h, w2_ref[...], preferred_element_type=jnp.float32)
    y2_ref[...] = y2 * dinv[:, None]


def _mm2(sp, y1, dinv, b1, w2):
    return pl.pallas_call(
        _mm2_body,
        grid=(_GRID,),
        in_specs=[
            pl.BlockSpec((NC, _BLK, D), lambda i: (0, i, 0)),
            pl.BlockSpec((_BLK, D), lambda i: (i, 0)),
            pl.BlockSpec((_BLK,), lambda i: (i,)),
            pl.BlockSpec((D,), lambda i: (0,)),
            pl.BlockSpec((D, D), lambda i: (0, 0)),
        ],
        out_specs=pl.BlockSpec((_BLK, D), lambda i: (i, 0)),
        out_shape=jax.ShapeDtypeStruct((N_PAD, D), jnp.float32),
    )(sp, y1, dinv, b1, w2)


def _out_body(sp_ref, y2_ref, dinv_ref, b2_ref, o_ref):
    sp = sp_ref[...]
    dinv = dinv_ref[...]
    s = (sp[0] + sp[1] + y2_ref[...]) * dinv[:, None] + b2_ref[...][None, :]
    m = jnp.max(s, axis=1, keepdims=True)
    lse = m + jnp.log(jnp.sum(jnp.exp(s - m), axis=1, keepdims=True))
    o_ref[...] = s - lse


def _outk(sp, y2, dinv, b2):
    return pl.pallas_call(
        _out_body,
        grid=(_GRID,),
        in_specs=[
            pl.BlockSpec((NC, _BLK, D), lambda i: (0, i, 0)),
            pl.BlockSpec((_BLK, D), lambda i: (i, 0)),
            pl.BlockSpec((_BLK,), lambda i: (i,)),
            pl.BlockSpec((D,), lambda i: (0,)),
        ],
        out_specs=pl.BlockSpec((_BLK, D), lambda i: (i, 0)),
        out_shape=jax.ShapeDtypeStruct((N_PAD, D), jnp.float32),
    )(sp, y2, dinv, b2)


def kernel(x, edge_index, W1, b1, W2, b2):
    e = edge_index.shape[1]
    e_pad = ((e + (NC * NS * CHUNK) - 1) // (NC * NS * CHUNK)) * (NC * NS * CHUNK)
    src = edge_index[0].astype(jnp.int32)
    dst = edge_index[1].astype(jnp.int32)
    src = jnp.concatenate([src, jnp.zeros((e_pad - e,), jnp.int32)])
    dst = jnp.concatenate([dst, jnp.full((e_pad - e,), N, jnp.int32)])
    src2 = src.reshape(e_pad // SUB, SUB)
    dst2 = dst.reshape(e_pad // SUB, SUB)
    x_pad = jnp.concatenate(
        [x.astype(jnp.float32), jnp.zeros((N_PAD - N, D), jnp.float32)]
    )
    iota2 = jnp.arange(N_PAD, dtype=jnp.int32).reshape(N_PAD // SUB, SUB)
    ones16 = jnp.ones((SUB, 16), jnp.float32)
    zer16 = jnp.zeros((SUB, 16), jnp.float32)
    zer128 = jnp.zeros((SUB, D), jnp.float32)

    degp = _deg_kernel(dst2, iota2, ones16, zer16).reshape(NC, N_PAD, 16)
    y1, dinv = _mm1(x_pad, W1.astype(jnp.float32), degp)
    s1p = _seg_kernel(y1, src2, dst2, iota2, zer128).reshape(NC, N_PAD, D)
    y2 = _mm2(s1p, y1, dinv, b1.astype(jnp.float32), W2.astype(jnp.float32))
    s2p = _seg_kernel(y2, src2, dst2, iota2, zer128).reshape(NC, N_PAD, D)
    out = _outk(s2p, y2, dinv, b2.astype(jnp.float32))
    return out[:N]


# speedup vs baseline: 9.1483x; 1.5385x over previous
import functools

import jax
import jax.numpy as jnp
from jax import lax
from jax.experimental import pallas as pl
from jax.experimental.pallas import tpu as pltpu
from jax.experimental.pallas import tpu_sc as plsc

N = 10000
D = 128
N_PAD = 10240
SUB = 16
CHUNK = 128
ROWS_PT = N_PAD // 16
NC, NS = 2, 16

_mesh = plsc.VectorSubcoreMesh(
    core_axis_name="c", subcore_axis_name="s", num_cores=NC, num_subcores=NS
)


def _deg_body(dst_hbm, iota_hbm, ones_hbm, zer_hbm, out_hbm,
              dst_v, idx_v, ones_v, zbuf_v, buf_v, acc, sem):
    cid = lax.axis_index("c")
    sid = lax.axis_index("s")
    wid = cid * NS + sid
    e_rows = dst_hbm.shape[0] // (NC * NS)
    nch = e_rows // (CHUNK // SUB)
    nz = ROWS_PT // CHUNK
    pltpu.async_copy(zer_hbm, zbuf_v, sem).wait()
    pltpu.async_copy(ones_hbm, ones_v, sem).wait()
    for k in range(nz):
        pltpu.async_copy(
            iota_hbm.at[pl.ds(sid * (ROWS_PT // SUB) + k * 8, 8)], idx_v, sem
        ).wait()
        for j in range(CHUNK // SUB):
            pltpu.async_copy(zbuf_v, acc.at[idx_v.at[j]], sem).wait()
    plsc.subcore_barrier()

    def body(c, carry):
        off = wid * e_rows + c * (CHUNK // SUB)
        pltpu.async_copy(dst_hbm.at[pl.ds(off, CHUNK // SUB)], dst_v, sem).wait()
        for j in range(CHUNK // SUB):
            pltpu.async_copy(ones_v, acc.at[dst_v.at[j]], sem, add=True).wait()
        return carry

    lax.fori_loop(0, nch, body, 0)
    plsc.subcore_barrier()
    for k in range(nz):
        pltpu.async_copy(
            iota_hbm.at[pl.ds(sid * (ROWS_PT // SUB) + k * 8, 8)], idx_v, sem
        ).wait()
        for j in range(CHUNK // SUB):
            pltpu.async_copy(acc.at[idx_v.at[j]],
                             buf_v.at[pl.ds(k * CHUNK + j * SUB, SUB)], sem).wait()
    pltpu.async_copy(
        buf_v, out_hbm.at[pl.ds(cid * N_PAD + sid * ROWS_PT, ROWS_PT)], sem
    ).wait()


@functools.partial(
    pl.kernel,
    out_type=jax.ShapeDtypeStruct((NC * N_PAD, 16), jnp.float32),
    mesh=_mesh,
    scratch_types=[
        pltpu.VMEM((CHUNK // SUB, SUB), jnp.int32),
        pltpu.VMEM((CHUNK // SUB, SUB), jnp.int32),
        pltpu.VMEM((SUB, 16), jnp.float32),
        pltpu.VMEM((SUB, 16), jnp.float32),
        pltpu.VMEM((ROWS_PT, 16), jnp.float32),
        pltpu.VMEM_SHARED((N_PAD, 16), jnp.float32),
        pltpu.SemaphoreType.DMA,
    ],
    compiler_params=pltpu.CompilerParams(use_tc_tiling_on_sc=False),
)
def _deg_kernel(dst_hbm, iota_hbm, ones_hbm, zer_hbm, out_hbm,
                dst_v, idx_v, ones_v, zbuf_v, buf_v, acc, sem):
    _deg_body(dst_hbm, iota_hbm, ones_hbm, zer_hbm, out_hbm,
              dst_v, idx_v, ones_v, zbuf_v, buf_v, acc, sem)


def _seg_body(y_hbm, src_hbm, dst_hbm, iota_hbm, zer_hbm, out_hbm,
              src_v, dst_v, idx_v, rows_v, zbuf_v, buf_v, acc, sem):
    cid = lax.axis_index("c")
    sid = lax.axis_index("s")
    wid = cid * NS + sid
    e_rows = src_hbm.shape[0] // (NC * NS)
    nch = e_rows // (CHUNK // SUB)
    nz = ROWS_PT // CHUNK
    pltpu.sync_copy(zer_hbm, zbuf_v)
    for k in range(nz):
        pltpu.sync_copy(iota_hbm.at[pl.ds(sid * (ROWS_PT // SUB) + k * 8, 8)], idx_v)
        for j in range(CHUNK // SUB):
            pltpu.sync_copy(zbuf_v, acc.at[idx_v.at[j]])
    plsc.subcore_barrier()

    def body(c, carry):
        off = wid * e_rows + c * (CHUNK // SUB)
        pltpu.sync_copy(src_hbm.at[pl.ds(off, CHUNK // SUB)], src_v)
        pltpu.sync_copy(dst_hbm.at[pl.ds(off, CHUNK // SUB)], dst_v)
        gat = [
            pltpu.async_copy(
                y_hbm.at[src_v.at[j]], buf_v.at[pl.ds(j * SUB, SUB)], sem
            )
            for j in range(CHUNK // SUB)
        ]
        for g in gat:
            g.wait()
        sca = [
            pltpu.async_copy(
                buf_v.at[pl.ds(j * SUB, SUB)], acc.at[dst_v.at[j]], sem, add=True
            )
            for j in range(CHUNK // SUB)
        ]
        for s in sca:
            s.wait()
        return carry

    lax.fori_loop(0, nch, body, 0)
    plsc.subcore_barrier()
    for k in range(nz):
        pltpu.sync_copy(iota_hbm.at[pl.ds(sid * (ROWS_PT // SUB) + k * 8, 8)], idx_v)
        for j in range(CHUNK // SUB):
            pltpu.sync_copy(acc.at[idx_v.at[j]],
                            buf_v.at[pl.ds(j * SUB, SUB)])
        pltpu.sync_copy(
            buf_v,
            out_hbm.at[pl.ds(cid * N_PAD + sid * ROWS_PT + k * CHUNK, CHUNK)],
        )


@functools.partial(
    pl.kernel,
    out_type=jax.ShapeDtypeStruct((NC * N_PAD, D), jnp.float32),
    mesh=_mesh,
    scratch_types=[
        pltpu.VMEM((CHUNK // SUB, SUB), jnp.int32),
        pltpu.VMEM((CHUNK // SUB, SUB), jnp.int32),
        pltpu.VMEM((CHUNK // SUB, SUB), jnp.int32),
        pltpu.VMEM((SUB, D), jnp.float32),
        pltpu.VMEM((SUB, D), jnp.float32),
        pltpu.VMEM((CHUNK, D), jnp.float32),
        pltpu.VMEM_SHARED((N_PAD, D), jnp.float32),
        pltpu.SemaphoreType.DMA,
    ],
)
def _seg_kernel(y_hbm, src_hbm, dst_hbm, iota_hbm, zer_hbm, out_hbm,
                src_v, dst_v, idx_v, rows_v, zbuf_v, buf_v, acc, sem):
    _seg_body(y_hbm, src_hbm, dst_hbm, iota_hbm, zer_hbm, out_hbm,
              src_v, dst_v, idx_v, rows_v, zbuf_v, buf_v, acc, sem)


_BLK = 1024
_GRID = N_PAD // _BLK


def _mm1_body(x_ref, w_ref, degp_ref, y_ref, dinv_ref):
    degp = degp_ref[...]
    deg = degp[0, :, 0] + degp[1, :, 0] + 1.0
    dinv = lax.rsqrt(deg)
    y = jnp.dot(x_ref[...], w_ref[...], preferred_element_type=jnp.float32)
    y_ref[...] = y * dinv[:, None]
    dinv_ref[...] = dinv


def _mm1(x, w1, degp):
    return pl.pallas_call(
        _mm1_body,
        grid=(_GRID,),
        in_specs=[
            pl.BlockSpec((_BLK, D), lambda i: (i, 0)),
            pl.BlockSpec((D, D), lambda i: (0, 0)),
            pl.BlockSpec((NC, _BLK, 16), lambda i: (0, i, 0)),
        ],
        out_specs=[
            pl.BlockSpec((_BLK, D), lambda i: (i, 0)),
            pl.BlockSpec((_BLK,), lambda i: (i,)),
        ],
        out_shape=[
            jax.ShapeDtypeStruct((N_PAD, D), jnp.float32),
            jax.ShapeDtypeStruct((N_PAD,), jnp.float32),
        ],
    )(x, w1, degp)


def _mm2_body(sp_ref, y1_ref, dinv_ref, b1_ref, w2_ref, y2_ref):
    sp = sp_ref[...]
    dinv = dinv_ref[...]
    s = sp[0] + sp[1] + y1_ref[...]
    h = jnp.maximum(s * dinv[:, None] + b1_ref[...][None, :], 0.0)
    y2 = jnp.dot(h, w2_ref[...], preferred_element_type=jnp.float32)
    y2_ref[...] = y2 * dinv[:, None]


def _mm2(sp, y1, dinv, b1, w2):
    return pl.pallas_call(
        _mm2_body,
        grid=(_GRID,),
        in_specs=[
            pl.BlockSpec((NC, _BLK, D), lambda i: (0, i, 0)),
            pl.BlockSpec((_BLK, D), lambda i: (i, 0)),
            pl.BlockSpec((_BLK,), lambda i: (i,)),
            pl.BlockSpec((D,), lambda i: (0,)),
            pl.BlockSpec((D, D), lambda i: (0, 0)),
        ],
        out_specs=pl.BlockSpec((_BLK, D), lambda i: (i, 0)),
        out_shape=jax.ShapeDtypeStruct((N_PAD, D), jnp.float32),
    )(sp, y1, dinv, b1, w2)


def _out_body(sp_ref, y2_ref, dinv_ref, b2_ref, o_ref):
    sp = sp_ref[...]
    dinv = dinv_ref[...]
    s = (sp[0] + sp[1] + y2_ref[...]) * dinv[:, None] + b2_ref[...][None, :]
    m = jnp.max(s, axis=1, keepdims=True)
    lse = m + jnp.log(jnp.sum(jnp.exp(s - m), axis=1, keepdims=True))
    o_ref[...] = s - lse


def _outk(sp, y2, dinv, b2):
    return pl.pallas_call(
        _out_body,
        grid=(_GRID,),
        in_specs=[
            pl.BlockSpec((NC, _BLK, D), lambda i: (0, i, 0)),
            pl.BlockSpec((_BLK, D), lambda i: (i, 0)),
            pl.BlockSpec((_BLK,), lambda i: (i,)),
            pl.BlockSpec((D,), lambda i: (0,)),
        ],
        out_specs=pl.BlockSpec((_BLK, D), lambda i: (i, 0)),
        out_shape=jax.ShapeDtypeStruct((N_PAD, D), jnp.float32),
    )(sp, y2, dinv, b2)


def kernel(x, edge_index, W1, b1, W2, b2):
    e = edge_index.shape[1]
    e_pad = ((e + (NC * NS * CHUNK) - 1) // (NC * NS * CHUNK)) * (NC * NS * CHUNK)
    src = edge_index[0].astype(jnp.int32)
    dst = edge_index[1].astype(jnp.int32)
    src = jnp.concatenate([src, jnp.zeros((e_pad - e,), jnp.int32)])
    dst = jnp.concatenate([dst, jnp.full((e_pad - e,), N, jnp.int32)])
    src2 = src.reshape(e_pad // SUB, SUB)
    dst2 = dst.reshape(e_pad // SUB, SUB)
    x_pad = jnp.concatenate(
        [x.astype(jnp.float32), jnp.zeros((N_PAD - N, D), jnp.float32)]
    )
    iota2 = jnp.arange(N_PAD, dtype=jnp.int32).reshape(N_PAD // SUB, SUB)
    ones16 = jnp.ones((SUB, 16), jnp.float32)
    zer16 = jnp.zeros((SUB, 16), jnp.float32)
    zer128 = jnp.zeros((SUB, D), jnp.float32)

    degp = _deg_kernel(dst2, iota2, ones16, zer16).reshape(NC, N_PAD, 16)
    y1, dinv = _mm1(x_pad, W1.astype(jnp.float32), degp)
    s1p = _seg_kernel(y1, src2, dst2, iota2, zer128).reshape(NC, N_PAD, D)
    y2 = _mm2(s1p, y1, dinv, b1.astype(jnp.float32), W2.astype(jnp.float32))
    s2p = _seg_kernel(y2, src2, dst2, iota2, zer128).reshape(NC, N_PAD, D)
    out = _outk(s2p, y2, dinv, b2.astype(jnp.float32))
    return out[:N]


# speedup vs baseline: 9.4473x; 1.0327x over previous
import functools

import jax
import jax.numpy as jnp
from jax import lax
from jax.experimental import pallas as pl
from jax.experimental.pallas import tpu as pltpu
from jax.experimental.pallas import tpu_sc as plsc

N = 10000
D = 128
N_PAD = 10240
SUB = 16
CHUNK = 128
ROWS_PT = N_PAD // 16
NC, NS = 2, 16

_mesh = plsc.VectorSubcoreMesh(
    core_axis_name="c", subcore_axis_name="s", num_cores=NC, num_subcores=NS
)


def _deg_body(dst_hbm, iota_hbm, ones_hbm, zer_hbm, out_hbm,
              dst_v, idx_v, ones_v, zbuf_v, buf_v, acc, sem):
    cid = lax.axis_index("c")
    sid = lax.axis_index("s")
    wid = cid * NS + sid
    e_rows = dst_hbm.shape[0] // (NC * NS)
    nch = e_rows // (CHUNK // SUB)
    nz = ROWS_PT // CHUNK
    pltpu.async_copy(zer_hbm, zbuf_v, sem).wait()
    pltpu.async_copy(ones_hbm, ones_v, sem).wait()
    for k in range(nz):
        pltpu.async_copy(
            iota_hbm.at[pl.ds(sid * (ROWS_PT // SUB) + k * 8, 8)], idx_v, sem
        ).wait()
        for j in range(CHUNK // SUB):
            pltpu.async_copy(zbuf_v, acc.at[idx_v.at[j]], sem).wait()
    plsc.subcore_barrier()

    def body(c, carry):
        off = wid * e_rows + c * (CHUNK // SUB)
        pltpu.async_copy(dst_hbm.at[pl.ds(off, CHUNK // SUB)], dst_v, sem).wait()
        sca = [
            pltpu.async_copy(ones_v, acc.at[dst_v.at[j]], sem, add=True)
            for j in range(CHUNK // SUB)
        ]
        for s in sca:
            s.wait()
        return carry

    lax.fori_loop(0, nch, body, 0)
    plsc.subcore_barrier()
    for k in range(nz):
        pltpu.async_copy(
            iota_hbm.at[pl.ds(sid * (ROWS_PT // SUB) + k * 8, 8)], idx_v, sem
        ).wait()
        for j in range(CHUNK // SUB):
            pltpu.async_copy(acc.at[idx_v.at[j]],
                             buf_v.at[pl.ds(k * CHUNK + j * SUB, SUB)], sem).wait()
    pltpu.async_copy(
        buf_v, out_hbm.at[pl.ds(cid * N_PAD + sid * ROWS_PT, ROWS_PT)], sem
    ).wait()


@functools.partial(
    pl.kernel,
    out_type=jax.ShapeDtypeStruct((NC * N_PAD, 16), jnp.float32),
    mesh=_mesh,
    scratch_types=[
        pltpu.VMEM((CHUNK // SUB, SUB), jnp.int32),
        pltpu.VMEM((CHUNK // SUB, SUB), jnp.int32),
        pltpu.VMEM((SUB, 16), jnp.float32),
        pltpu.VMEM((SUB, 16), jnp.float32),
        pltpu.VMEM((ROWS_PT, 16), jnp.float32),
        pltpu.VMEM_SHARED((N_PAD, 16), jnp.float32),
        pltpu.SemaphoreType.DMA,
    ],
    compiler_params=pltpu.CompilerParams(use_tc_tiling_on_sc=False),
)
def _deg_kernel(dst_hbm, iota_hbm, ones_hbm, zer_hbm, out_hbm,
                dst_v, idx_v, ones_v, zbuf_v, buf_v, acc, sem):
    _deg_body(dst_hbm, iota_hbm, ones_hbm, zer_hbm, out_hbm,
              dst_v, idx_v, ones_v, zbuf_v, buf_v, acc, sem)


def _seg_body(y_hbm, src_hbm, dst_hbm, iota_hbm, zer_hbm, out_hbm,
              src_v, dst_v, idx_v, rows_v, zbuf_v, buf_v, acc, sem):
    cid = lax.axis_index("c")
    sid = lax.axis_index("s")
    wid = cid * NS + sid
    e_rows = src_hbm.shape[0] // (NC * NS)
    nch = e_rows // (CHUNK // SUB)
    nz = ROWS_PT // CHUNK
    pltpu.sync_copy(zer_hbm, zbuf_v)
    for k in range(nz):
        pltpu.sync_copy(iota_hbm.at[pl.ds(sid * (ROWS_PT // SUB) + k * 8, 8)], idx_v)
        for j in range(CHUNK // SUB):
            pltpu.sync_copy(zbuf_v, acc.at[idx_v.at[j]])
    plsc.subcore_barrier()

    def body(c, carry):
        off = wid * e_rows + c * (CHUNK // SUB)
        pltpu.sync_copy(src_hbm.at[pl.ds(off, CHUNK // SUB)], src_v)
        pltpu.sync_copy(dst_hbm.at[pl.ds(off, CHUNK // SUB)], dst_v)
        gat = [
            pltpu.async_copy(
                y_hbm.at[src_v.at[j]], buf_v.at[pl.ds(j * SUB, SUB)], sem
            )
            for j in range(CHUNK // SUB)
        ]
        for g in gat:
            g.wait()
        sca = [
            pltpu.async_copy(
                buf_v.at[pl.ds(j * SUB, SUB)], acc.at[dst_v.at[j]], sem, add=True
            )
            for j in range(CHUNK // SUB)
        ]
        for s in sca:
            s.wait()
        return carry

    lax.fori_loop(0, nch, body, 0)
    plsc.subcore_barrier()
    for k in range(nz):
        pltpu.sync_copy(iota_hbm.at[pl.ds(sid * (ROWS_PT // SUB) + k * 8, 8)], idx_v)
        for j in range(CHUNK // SUB):
            pltpu.sync_copy(acc.at[idx_v.at[j]],
                            buf_v.at[pl.ds(j * SUB, SUB)])
        pltpu.sync_copy(
            buf_v,
            out_hbm.at[pl.ds(cid * N_PAD + sid * ROWS_PT + k * CHUNK, CHUNK)],
        )


@functools.partial(
    pl.kernel,
    out_type=jax.ShapeDtypeStruct((NC * N_PAD, D), jnp.float32),
    mesh=_mesh,
    scratch_types=[
        pltpu.VMEM((CHUNK // SUB, SUB), jnp.int32),
        pltpu.VMEM((CHUNK // SUB, SUB), jnp.int32),
        pltpu.VMEM((CHUNK // SUB, SUB), jnp.int32),
        pltpu.VMEM((SUB, D), jnp.float32),
        pltpu.VMEM((SUB, D), jnp.float32),
        pltpu.VMEM((CHUNK, D), jnp.float32),
        pltpu.VMEM_SHARED((N_PAD, D), jnp.float32),
        pltpu.SemaphoreType.DMA,
    ],
)
def _seg_kernel(y_hbm, src_hbm, dst_hbm, iota_hbm, zer_hbm, out_hbm,
                src_v, dst_v, idx_v, rows_v, zbuf_v, buf_v, acc, sem):
    _seg_body(y_hbm, src_hbm, dst_hbm, iota_hbm, zer_hbm, out_hbm,
              src_v, dst_v, idx_v, rows_v, zbuf_v, buf_v, acc, sem)


_BLK = 1024
_GRID = N_PAD // _BLK


def _mm1_body(x_ref, w_ref, degp_ref, y_ref, dinv_ref):
    degp = degp_ref[...]
    deg = degp[0, :, 0] + degp[1, :, 0] + 1.0
    dinv = lax.rsqrt(deg)
    y = jnp.dot(x_ref[...], w_ref[...], preferred_element_type=jnp.float32)
    y_ref[...] = y * dinv[:, None]
    dinv_ref[...] = dinv


def _mm1(x, w1, degp):
    return pl.pallas_call(
        _mm1_body,
        grid=(_GRID,),
        in_specs=[
            pl.BlockSpec((_BLK, D), lambda i: (i, 0)),
            pl.BlockSpec((D, D), lambda i: (0, 0)),
            pl.BlockSpec((NC, _BLK, 16), lambda i: (0, i, 0)),
        ],
        out_specs=[
            pl.BlockSpec((_BLK, D), lambda i: (i, 0)),
            pl.BlockSpec((_BLK,), lambda i: (i,)),
        ],
        out_shape=[
            jax.ShapeDtypeStruct((N_PAD, D), jnp.float32),
            jax.ShapeDtypeStruct((N_PAD,), jnp.float32),
        ],
    )(x, w1, degp)


def _mm2_body(sp_ref, y1_ref, dinv_ref, b1_ref, w2_ref, y2_ref):
    sp = sp_ref[...]
    dinv = dinv_ref[...]
    s = sp[0] + sp[1] + y1_ref[...]
    h = jnp.maximum(s * dinv[:, None] + b1_ref[...][None, :], 0.0)
    y2 = jnp.dot(h, w2_ref[...], preferred_element_type=jnp.float32)
    y2_ref[...] = y2 * dinv[:, None]


def _mm2(sp, y1, dinv, b1, w2):
    return pl.pallas_call(
        _mm2_body,
        grid=(_GRID,),
        in_specs=[
            pl.BlockSpec((NC, _BLK, D), lambda i: (0, i, 0)),
            pl.BlockSpec((_BLK, D), lambda i: (i, 0)),
            pl.BlockSpec((_BLK,), lambda i: (i,)),
            pl.BlockSpec((D,), lambda i: (0,)),
            pl.BlockSpec((D, D), lambda i: (0, 0)),
        ],
        out_specs=pl.BlockSpec((_BLK, D), lambda i: (i, 0)),
        out_shape=jax.ShapeDtypeStruct((N_PAD, D), jnp.float32),
    )(sp, y1, dinv, b1, w2)


def _out_body(sp_ref, y2_ref, dinv_ref, b2_ref, o_ref):
    sp = sp_ref[...]
    dinv = dinv_ref[...]
    s = (sp[0] + sp[1] + y2_ref[...]) * dinv[:, None] + b2_ref[...][None, :]
    m = jnp.max(s, axis=1, keepdims=True)
    lse = m + jnp.log(jnp.sum(jnp.exp(s - m), axis=1, keepdims=True))
    o_ref[...] = s - lse


def _outk(sp, y2, dinv, b2):
    return pl.pallas_call(
        _out_body,
        grid=(_GRID,),
        in_specs=[
            pl.BlockSpec((NC, _BLK, D), lambda i: (0, i, 0)),
            pl.BlockSpec((_BLK, D), lambda i: (i, 0)),
            pl.BlockSpec((_BLK,), lambda i: (i,)),
            pl.BlockSpec((D,), lambda i: (0,)),
        ],
        out_specs=pl.BlockSpec((_BLK, D), lambda i: (i, 0)),
        out_shape=jax.ShapeDtypeStruct((N_PAD, D), jnp.float32),
    )(sp, y2, dinv, b2)


def kernel(x, edge_index, W1, b1, W2, b2):
    e = edge_index.shape[1]
    e_pad = ((e + (NC * NS * CHUNK) - 1) // (NC * NS * CHUNK)) * (NC * NS * CHUNK)
    src = edge_index[0].astype(jnp.int32)
    dst = edge_index[1].astype(jnp.int32)
    src = jnp.concatenate([src, jnp.zeros((e_pad - e,), jnp.int32)])
    dst = jnp.concatenate([dst, jnp.full((e_pad - e,), N, jnp.int32)])
    src2 = src.reshape(e_pad // SUB, SUB)
    dst2 = dst.reshape(e_pad // SUB, SUB)
    x_pad = jnp.concatenate(
        [x.astype(jnp.float32), jnp.zeros((N_PAD - N, D), jnp.float32)]
    )
    iota2 = jnp.arange(N_PAD, dtype=jnp.int32).reshape(N_PAD // SUB, SUB)
    ones16 = jnp.ones((SUB, 16), jnp.float32)
    zer16 = jnp.zeros((SUB, 16), jnp.float32)
    zer128 = jnp.zeros((SUB, D), jnp.float32)

    degp = _deg_kernel(dst2, iota2, ones16, zer16).reshape(NC, N_PAD, 16)
    y1, dinv = _mm1(x_pad, W1.astype(jnp.float32), degp)
    s1p = _seg_kernel(y1, src2, dst2, iota2, zer128).reshape(NC, N_PAD, D)
    y2 = _mm2(s1p, y1, dinv, b1.astype(jnp.float32), W2.astype(jnp.float32))
    s2p = _seg_kernel(y2, src2, dst2, iota2, zer128).reshape(NC, N_PAD, D)
    out = _outk(s2p, y2, dinv, b2.astype(jnp.float32))
    return out[:N]
